# baseline (speedup 1.0000x reference)
import functools

import jax
import jax.numpy as jnp
from jax import lax
from jax.experimental import pallas as pl
from jax.experimental.pallas import tpu as pltpu
from jax.experimental.pallas import tpu_sc as plsc

BATCH = 16384
EMB = 64
K = 128

NC = 2
NS = 16
NW = NC * NS
BPW = BATCH // NW
CH = 128
NCH = BPW // CH


def _sc_gather_body(uid_hbm, iid_hbm, mfu_hbm, mfi_hbm, mlpu_hbm, mlpi_hbm,
                    xmf_out, xu_out, xi_out,
                    idx_u, idx_i, buf_a, buf_b, sem):
    wid = lax.axis_index("s") * NC + lax.axis_index("c")
    pltpu.sync_copy(uid_hbm.at[wid], idx_u)
    pltpu.sync_copy(iid_hbm.at[wid], idx_i)

    handles = []
    for j in range(NCH):
        handles.append(pltpu.async_copy(mfu_hbm.at[idx_u.at[j]],
                                        buf_a.at[pl.ds(j * CH, CH)], sem))
        handles.append(pltpu.async_copy(mfi_hbm.at[idx_i.at[j]],
                                        buf_b.at[pl.ds(j * CH, CH)], sem))
    for h in handles:
        h.wait()

    def prod_row(r, carry):
        for c in range(EMB // 16):
            sl = pl.ds(c * 16, 16)
            buf_a[r, sl] = buf_a[r, sl] * buf_b[r, sl]
        return carry
    lax.fori_loop(0, BPW, prod_row, 0)
    pltpu.sync_copy(buf_a, xmf_out.at[wid])

    handles = []
    for j in range(NCH):
        handles.append(pltpu.async_copy(mlpu_hbm.at[idx_u.at[j]],
                                        buf_a.at[pl.ds(j * CH, CH)], sem))
        handles.append(pltpu.async_copy(mlpi_hbm.at[idx_i.at[j]],
                                        buf_b.at[pl.ds(j * CH, CH)], sem))
    for h in handles:
        h.wait()
    pltpu.sync_copy(buf_a, xu_out.at[wid])
    pltpu.sync_copy(buf_b, xi_out.at[wid])


_sc_gather = functools.partial(
    pl.kernel,
    mesh=plsc.VectorSubcoreMesh(core_axis_name="c", subcore_axis_name="s"),
    out_type=(
        jax.ShapeDtypeStruct((NW, BPW, EMB), jnp.float32),
        jax.ShapeDtypeStruct((NW, BPW, EMB), jnp.float32),
        jax.ShapeDtypeStruct((NW, BPW, EMB), jnp.float32),
    ),
    scratch_types=[
        pltpu.VMEM((NCH, CH), jnp.int32),
        pltpu.VMEM((NCH, CH), jnp.int32),
        pltpu.VMEM((BPW, EMB), jnp.float32),
        pltpu.VMEM((BPW, EMB), jnp.float32),
        pltpu.SemaphoreType.DMA,
    ],
    compiler_params=pltpu.CompilerParams(use_tc_tiling_on_sc=False),
)(_sc_gather_body)


def _tc_mlp_body(xmf_ref, xu_ref, xi_ref, w1a_ref, w1b_ref, b1_ref,
                 w2_ref, b2_ref, wa_ref, wb_ref, bout_ref, out_ref):
    f32 = jnp.float32
    h = jnp.dot(xu_ref[...], w1a_ref[...], preferred_element_type=f32)
    h += jnp.dot(xi_ref[...], w1b_ref[...], preferred_element_type=f32)
    h = jnp.maximum(h + b1_ref[...], 0.0)
    h = jnp.dot(h, w2_ref[...], preferred_element_type=f32)
    h = jnp.maximum(h + b2_ref[...], 0.0)
    z = jnp.dot(xmf_ref[...], wa_ref[...], preferred_element_type=f32)
    z += jnp.dot(h, wb_ref[...], preferred_element_type=f32)
    z += bout_ref[...]
    out_ref[...] = jnp.where(z > 0.0, z, jnp.exp(z) - 1.0)


def kernel(user_id, item_id, mf_user, mf_item, mlp_user, mlp_item,
           W1, b1, W2, b2, Wout, bout):
    uid = user_id.astype(jnp.int32).reshape(NW, NCH, CH)
    iid = item_id.astype(jnp.int32).reshape(NW, NCH, CH)
    xmf, xu, xi = _sc_gather(uid, iid, mf_user, mf_item, mlp_user, mlp_item)
    xmf = xmf.reshape(BATCH, EMB)
    xu = xu.reshape(BATCH, EMB)
    xi = xi.reshape(BATCH, EMB)

    BLK = 2048
    grid = (BATCH // BLK,)
    zero = lambda i: (0, 0)
    out = pl.pallas_call(
        _tc_mlp_body,
        grid=grid,
        in_specs=[
            pl.BlockSpec((BLK, EMB), lambda i: (i, 0)),
            pl.BlockSpec((BLK, EMB), lambda i: (i, 0)),
            pl.BlockSpec((BLK, EMB), lambda i: (i, 0)),
            pl.BlockSpec((EMB, K), zero),
            pl.BlockSpec((EMB, K), zero),
            pl.BlockSpec((1, K), zero),
            pl.BlockSpec((K, K), zero),
            pl.BlockSpec((1, K), zero),
            pl.BlockSpec((EMB, 1), zero),
            pl.BlockSpec((K, 1), zero),
            pl.BlockSpec((1, 1), zero),
        ],
        out_specs=pl.BlockSpec((BLK, 1), lambda i: (i, 0)),
        out_shape=jax.ShapeDtypeStruct((BATCH, 1), jnp.float32),
    )(
        xmf, xu, xi,
        W1[:EMB, :], W1[EMB:, :], b1.reshape(1, K),
        W2, b2.reshape(1, K),
        Wout[:EMB, :], Wout[EMB:, :], bout.reshape(1, 1),
    )
    return out
